# baseline (speedup 1.0000x reference)
import functools

import jax
import jax.numpy as jnp
from jax import lax
from jax.experimental import pallas as pl
from jax.experimental.pallas import tpu as pltpu
from jax.experimental.pallas import tpu_sc as plsc

F32 = jnp.float32
I32 = jnp.int32

K = 20
EPS = 1e-5
R = 256
BIG = 3.0e38


def _knn_body(f_full_ref, f_blk_ref, w1_ref, w2_ref,
              aT_ref, eT_ref, idx_ref, pd_ref):
    b = pl.program_id(0)
    n_total = f_full_ref.shape[2]

    fb = f_blk_ref[0]
    w1 = w1_ref[...]
    w2 = w2_ref[...]

    dn = (((0,), (1,)), ((), ()))
    localT = lax.dot_general(fb, w1, dn, preferred_element_type=F32)
    edgeT = lax.dot_general(fb, w2, dn, preferred_element_type=F32)
    aT_ref[0] = localT + edgeT
    eT_ref[0] = jnp.concatenate(
        [edgeT, jnp.zeros((edgeT.shape[0], 128 - edgeT.shape[1]), F32)],
        axis=1)

    ff = f_full_ref[0]
    inner = lax.dot_general(fb, ff, (((0,), (0,)), ((), ())),
                            preferred_element_type=F32)
    sqf = jnp.sum(ff * ff, axis=0, keepdims=True)

    base = b * n_total

    NL = 128
    RH = 128
    steps = n_total // NL
    BIGI = 2 ** 30
    nfail = 0
    nh = R // RH
    lane = lax.broadcasted_iota(I32, (RH, NL), 1)
    halves = []
    for rh in range(nh):
        rs = slice(rh * RH, (rh + 1) * RH)
        chain = [[jnp.full((RH, NL), BIG, F32), jnp.full((RH, NL), BIGI, I32)]
                 for _ in range(4)]
        dmin = jnp.full((RH, NL), BIG, F32)
        for s in range(steps):
            sl = slice(s * NL, (s + 1) * NL)
            t = sqf[:, sl] - 2.0 * inner[rs, sl]
            it = lane + (s * NL)
            for lv in range(4):
                mv, mi = chain[lv]
                c = t < mv
                chain[lv][0] = jnp.minimum(mv, t)
                chain[lv][1] = jnp.where(c, it, mi)
                t, it = jnp.maximum(mv, t), jnp.where(c, mi, it)
            dmin = jnp.minimum(dmin, t)
        halves.append([chain, dmin])

    for j in range(K):
        for rh in range(nh):
            chain = halves[rh][0]
            (m1, i1), (m2, i2), (m3, i3), (m4, i4) = chain
            mval = jnp.min(m1, axis=1, keepdims=True)
            selidx = jnp.min(jnp.where(m1 == mval, i1, BIGI),
                             axis=1, keepdims=True)
            idx_ref[0, rh * RH:(rh + 1) * RH, j:j + 1] = selidx + base
            pm = i1 == selidx
            chain[0] = [jnp.where(pm, m2, m1), jnp.where(pm, i2, i1)]
            chain[1] = [jnp.where(pm, m3, m2), jnp.where(pm, i3, i2)]
            chain[2] = [jnp.where(pm, m4, m3), jnp.where(pm, i4, i3)]
            chain[3] = [jnp.where(pm, BIG, m4), jnp.where(pm, BIGI, i4)]
            if j == K - 1:
                dm_row = jnp.min(halves[rh][1], axis=1, keepdims=True)
                nfail = nfail + jnp.max(jnp.where(dm_row <= mval, 1, 0))

    @pl.when(nfail > 0)
    def _slow_exact():
        iota = lax.broadcasted_iota(I32, (R, n_total), 1)
        pd_ref[...] = sqf - 2.0 * inner
        for j in range(K):
            pd = pd_ref[...]
            m = jnp.min(pd, axis=1, keepdims=True)
            tie = jnp.where(pd == m, iota, n_total)
            sel = jnp.min(tie, axis=1, keepdims=True)
            idx_ref[0, :, j:j + 1] = sel + base
            pd_ref[...] = jnp.where(iota == sel, BIG, pd)


def _knn_topk(features, W1, W2):
    B, C, N = features.shape
    nb = N // R
    grid = (B, nb)
    out_shapes = [
        jax.ShapeDtypeStruct((B, N, C), F32),
        jax.ShapeDtypeStruct((B, N, 128), F32),
        jax.ShapeDtypeStruct((B, N, K), I32),
    ]
    return pl.pallas_call(
        _knn_body,
        grid=grid,
        in_specs=[
            pl.BlockSpec((1, C, N), lambda b, i: (b, 0, 0)),
            pl.BlockSpec((1, C, R), lambda b, i: (b, 0, i)),
            pl.BlockSpec((C, C), lambda b, i: (0, 0)),
            pl.BlockSpec((C, C), lambda b, i: (0, 0)),
        ],
        out_specs=[
            pl.BlockSpec((1, R, C), lambda b, i: (b, i, 0)),
            pl.BlockSpec((1, R, 128), lambda b, i: (b, i, 0)),
            pl.BlockSpec((1, R, K), lambda b, i: (b, i, 0)),
        ],
        out_shape=out_shapes,
        scratch_shapes=[pltpu.VMEM((R, N), F32)],
        compiler_params=pltpu.CompilerParams(
            dimension_semantics=("parallel", "parallel")),
    )(features, features, W1, W2)


def _sc_gather_stats(edgeT2d, idx3d, aT2w, nw, rows_w, chunks, rows_ck):
    BN, CP = edgeT2d.shape
    C = CP // 2
    idx_ck = rows_ck * K
    nbuf = 8
    mesh = plsc.VectorSubcoreMesh(core_axis_name="c", subcore_axis_name="s")

    @functools.partial(
        pl.kernel,
        mesh=mesh,
        out_type=[
            jax.ShapeDtypeStruct((BN, CP), F32),
            jax.ShapeDtypeStruct((nw, 2, C), F32),
        ],
        scratch_types=[
            pltpu.VMEM((chunks, idx_ck), I32),
            pltpu.VMEM((rows_w // 2, CP), F32),
        ] + [pltpu.VMEM((idx_ck, CP), F32) for _ in range(nbuf)]
          + [pltpu.VMEM((rows_w, CP), F32),
             pltpu.VMEM((2, C), F32)]
          + [pltpu.SemaphoreType.DMA for _ in range(nbuf)],
    )
    def run(edge_hbm, idx_hbm, a_hbm, gmm_hbm, part_hbm,
            idx_v, a_v, *rest):
        bufs = rest[:nbuf]
        gmm_v, p_v = rest[nbuf], rest[nbuf + 1]
        sems = rest[nbuf + 2:]
        wid = lax.axis_index("s") * 2 + lax.axis_index("c")
        base = wid * rows_w

        pltpu.sync_copy(idx_hbm.at[wid], idx_v)
        pltpu.sync_copy(a_hbm.at[pl.ds(wid * (rows_w // 2), rows_w // 2)], a_v)

        zeros16 = jnp.zeros((16,), F32)
        for cc in range(C // 16):
            p_v[0, cc * 16:(cc + 1) * 16] = zeros16
            p_v[1, cc * 16:(cc + 1) * 16] = zeros16

        def group_body(t, _):
            g0 = t * nbuf
            handles = [
                pltpu.async_copy(edge_hbm.at[idx_v.at[g0 + s]], bufs[s], sems[s])
                for s in range(nbuf)
            ]
            for s in range(nbuf):
                g = g0 + s
                handles[s].wait()
                rows_v = bufs[s]

                def row_body(r, _, rows_v=rows_v, g=g):
                    row = g * rows_ck + r
                    for cc in range(C // 16):
                        sl = slice(cc * 16, (cc + 1) * 16)
                        v0 = rows_v[r * K, sl]
                        s_acc = v0
                        q_acc = v0 * v0
                        mn = v0
                        mx = v0
                        for j in range(1, K):
                            v = rows_v[r * K + j, sl]
                            s_acc = s_acc + v
                            q_acc = q_acc + v * v
                            mn = jnp.minimum(mn, v)
                            mx = jnp.maximum(mx, v)
                        gmm_v[row, sl] = mn
                        gmm_v[row, C + cc * 16:C + (cc + 1) * 16] = mx
                        a = a_v[row // 2, pl.ds((row % 2) * C + cc * 16, 16)]
                        p_v[0, sl] = p_v[0, sl] + (K * a - s_acc)
                        p_v[1, sl] = (p_v[1, sl]
                                      + (K * (a * a) - 2.0 * (a * s_acc) + q_acc))
                    return _

                lax.fori_loop(0, rows_ck, row_body, 0)
            return _

        lax.fori_loop(0, chunks // nbuf, group_body, 0)

        pltpu.sync_copy(gmm_v, gmm_hbm.at[pl.ds(base, rows_w)])
        pltpu.sync_copy(p_v, part_hbm.at[wid])

    return run(edgeT2d, idx3d, aT2w)


def _final_body(cnt, aT_ref, gmm_ref, part_ref, gamma_ref, beta_ref, out_ref):
    p = jnp.sum(part_ref[...], axis=0)
    mean = p[0:1] / cnt
    var = p[1:2] / cnt - mean * mean
    inv = 1.0 / jnp.sqrt(var + EPS)
    s = gamma_ref[...] * inv
    bias = beta_ref[...] - mean * s

    C = aT_ref.shape[2]
    aT = aT_ref[0]
    gsel = jnp.where(s >= 0.0, gmm_ref[0, :, :C], gmm_ref[0, :, C:])
    y = (aT - gsel) * s + bias
    y = jnp.maximum(y, 0.0)
    out_ref[0] = jnp.transpose(y)


def _finalize(aT, gmm3d, part, gamma2d, beta2d):
    B, N, C = aT.shape
    nw = part.shape[0]
    nb = N // R
    cnt = float(B * N * K)
    return pl.pallas_call(
        functools.partial(_final_body, cnt),
        grid=(B, nb),
        in_specs=[
            pl.BlockSpec((1, R, C), lambda b, i: (b, i, 0)),
            pl.BlockSpec((1, R, 2 * C), lambda b, i: (b, i, 0)),
            pl.BlockSpec((nw, 2, C), lambda b, i: (0, 0, 0)),
            pl.BlockSpec((1, C), lambda b, i: (0, 0)),
            pl.BlockSpec((1, C), lambda b, i: (0, 0)),
        ],
        out_specs=pl.BlockSpec((1, C, R), lambda b, i: (b, 0, i)),
        out_shape=jax.ShapeDtypeStruct((B, C, N), F32),
        compiler_params=pltpu.CompilerParams(
            dimension_semantics=("parallel", "parallel")),
    )(aT, gmm3d, part, gamma2d, beta2d)


def kernel(features, W1, W2, gamma, beta):
    B, C, N = features.shape
    BN = B * N

    aT, edgeT, idx = _knn_topk(features, W1, W2)

    info = plsc.get_sparse_core_info()
    nw = info.num_cores * info.num_subcores
    rows_w = BN // nw
    rows_ck = 2
    chunks = rows_w // rows_ck
    idx3d = idx.reshape(nw, chunks, rows_ck * K)

    gmm, part = _sc_gather_stats(
        edgeT.reshape(BN, 128), idx3d, aT.reshape(BN // 2, 2 * C),
        nw, rows_w, chunks, rows_ck)

    return _finalize(aT, gmm.reshape(B, N, 2 * C), part,
                     gamma[None, :], beta[None, :])


# speedup vs baseline: 1.0178x; 1.0178x over previous
import functools

import jax
import jax.numpy as jnp
from jax import lax
from jax.experimental import pallas as pl
from jax.experimental.pallas import tpu as pltpu
from jax.experimental.pallas import tpu_sc as plsc

F32 = jnp.float32
I32 = jnp.int32

K = 20
EPS = 1e-5
R = 256
BIG = 3.0e38


def _knn_body(f_full_ref, f_blk_ref, w1_ref, w2_ref,
              aT_ref, eT_ref, idx_ref, pd_ref):
    b = pl.program_id(0)
    n_total = f_full_ref.shape[2]

    fb = f_blk_ref[0]
    w1 = w1_ref[...]
    w2 = w2_ref[...]

    dn = (((0,), (1,)), ((), ()))
    localT = lax.dot_general(fb, w1, dn, preferred_element_type=F32)
    edgeT = lax.dot_general(fb, w2, dn, preferred_element_type=F32)
    aT_ref[0] = localT + edgeT
    eT_ref[0] = jnp.concatenate(
        [edgeT, jnp.zeros((edgeT.shape[0], 128 - edgeT.shape[1]), F32)],
        axis=1)

    ff = f_full_ref[0]
    inner = lax.dot_general(fb, ff, (((0,), (0,)), ((), ())),
                            preferred_element_type=F32)
    sqf = jnp.sum(ff * ff, axis=0, keepdims=True)

    base = b * n_total

    NL = 128
    RH = 128
    steps = n_total // NL
    BIGI = 2 ** 30
    nfail = 0
    nh = R // RH
    lane = lax.broadcasted_iota(I32, (RH, NL), 1)
    halves = []
    for rh in range(nh):
        rs = slice(rh * RH, (rh + 1) * RH)
        chain = [[jnp.full((RH, NL), BIG, F32), jnp.full((RH, NL), BIGI, I32)]
                 for _ in range(4)]
        dmin = jnp.full((RH, NL), BIG, F32)
        for s in range(steps):
            sl = slice(s * NL, (s + 1) * NL)
            t = sqf[:, sl] - 2.0 * inner[rs, sl]
            it = lane + (s * NL)
            for lv in range(4):
                mv, mi = chain[lv]
                c = t < mv
                chain[lv][0] = jnp.minimum(mv, t)
                chain[lv][1] = jnp.where(c, it, mi)
                t, it = jnp.maximum(mv, t), jnp.where(c, mi, it)
            dmin = jnp.minimum(dmin, t)
        halves.append([chain, dmin])

    for j in range(K):
        for rh in range(nh):
            chain = halves[rh][0]
            (m1, i1), (m2, i2), (m3, i3), (m4, i4) = chain
            mval = jnp.min(m1, axis=1, keepdims=True)
            selidx = jnp.min(jnp.where(m1 == mval, i1, BIGI),
                             axis=1, keepdims=True)
            idx_ref[0, rh * RH:(rh + 1) * RH, j:j + 1] = selidx + base
            pm = i1 == selidx
            chain[0] = [jnp.where(pm, m2, m1), jnp.where(pm, i2, i1)]
            chain[1] = [jnp.where(pm, m3, m2), jnp.where(pm, i3, i2)]
            chain[2] = [jnp.where(pm, m4, m3), jnp.where(pm, i4, i3)]
            chain[3] = [jnp.where(pm, BIG, m4), jnp.where(pm, BIGI, i4)]
            if j == K - 1:
                dm_row = jnp.min(halves[rh][1], axis=1, keepdims=True)
                nfail = nfail + jnp.max(jnp.where(dm_row <= mval, 1, 0))

    @pl.when(nfail > 0)
    def _slow_exact():
        iota = lax.broadcasted_iota(I32, (R, n_total), 1)
        pd_ref[...] = sqf - 2.0 * inner
        for j in range(K):
            pd = pd_ref[...]
            m = jnp.min(pd, axis=1, keepdims=True)
            tie = jnp.where(pd == m, iota, n_total)
            sel = jnp.min(tie, axis=1, keepdims=True)
            idx_ref[0, :, j:j + 1] = sel + base
            pd_ref[...] = jnp.where(iota == sel, BIG, pd)


def _knn_topk(features, W1, W2):
    B, C, N = features.shape
    nb = N // R
    grid = (B, nb)
    out_shapes = [
        jax.ShapeDtypeStruct((B, N, C), F32),
        jax.ShapeDtypeStruct((B, N, 128), F32),
        jax.ShapeDtypeStruct((B, N, K), I32),
    ]
    return pl.pallas_call(
        _knn_body,
        grid=grid,
        in_specs=[
            pl.BlockSpec((1, C, N), lambda b, i: (b, 0, 0)),
            pl.BlockSpec((1, C, R), lambda b, i: (b, 0, i)),
            pl.BlockSpec((C, C), lambda b, i: (0, 0)),
            pl.BlockSpec((C, C), lambda b, i: (0, 0)),
        ],
        out_specs=[
            pl.BlockSpec((1, R, C), lambda b, i: (b, i, 0)),
            pl.BlockSpec((1, R, 128), lambda b, i: (b, i, 0)),
            pl.BlockSpec((1, R, K), lambda b, i: (b, i, 0)),
        ],
        out_shape=out_shapes,
        scratch_shapes=[pltpu.VMEM((R, N), F32)],
        compiler_params=pltpu.CompilerParams(
            dimension_semantics=("parallel", "parallel")),
    )(features, features, W1, W2)


def _sc_gather_stats(edgeT2d, idx3d, aT2w, nw, rows_w, chunks, rows_ck):
    BN, CP = edgeT2d.shape
    C = CP // 2
    idx_ck = rows_ck * K
    nbuf = 4
    mesh = plsc.VectorSubcoreMesh(core_axis_name="c", subcore_axis_name="s")

    @functools.partial(
        pl.kernel,
        mesh=mesh,
        out_type=[
            jax.ShapeDtypeStruct((BN, CP), F32),
            jax.ShapeDtypeStruct((nw, 2, C), F32),
        ],
        scratch_types=[
            pltpu.VMEM((chunks, idx_ck), I32),
            pltpu.VMEM((rows_w // 2, CP), F32),
            pltpu.VMEM((idx_ck, CP), F32),
            pltpu.VMEM((idx_ck, CP), F32),
            pltpu.VMEM((idx_ck, CP), F32),
            pltpu.VMEM((idx_ck, CP), F32),
            pltpu.VMEM((rows_w, CP), F32),
            pltpu.VMEM((2, C), F32),
            pltpu.SemaphoreType.DMA,
            pltpu.SemaphoreType.DMA,
            pltpu.SemaphoreType.DMA,
            pltpu.SemaphoreType.DMA,
        ],
    )
    def run(edge_hbm, idx_hbm, a_hbm, gmm_hbm, part_hbm,
            idx_v, a_v, r0, r1, r2, r3, gmm_v, p_v, s0, s1, s2, s3):
        wid = lax.axis_index("s") * 2 + lax.axis_index("c")
        base = wid * rows_w
        bufs = (r0, r1, r2, r3)
        sems = (s0, s1, s2, s3)

        pltpu.sync_copy(idx_hbm.at[wid], idx_v)
        pltpu.sync_copy(a_hbm.at[pl.ds(wid * (rows_w // 2), rows_w // 2)], a_v)

        zeros16 = jnp.zeros((16,), F32)
        for cc in range(C // 16):
            p_v[0, cc * 16:(cc + 1) * 16] = zeros16
            p_v[1, cc * 16:(cc + 1) * 16] = zeros16

        def group_body(t, _):
            g0 = t * nbuf
            handles = [
                pltpu.async_copy(edge_hbm.at[idx_v.at[g0 + s]], bufs[s], sems[s])
                for s in range(nbuf)
            ]
            for s in range(nbuf):
                g = g0 + s
                handles[s].wait()
                rows_v = bufs[s]

                def row_body(r, _, rows_v=rows_v, g=g):
                    row = g * rows_ck + r
                    for cc in range(C // 16):
                        sl = slice(cc * 16, (cc + 1) * 16)
                        v0 = rows_v[r * K, sl]
                        s_acc = v0
                        q_acc = v0 * v0
                        mn = v0
                        mx = v0
                        for j in range(1, K):
                            v = rows_v[r * K + j, sl]
                            s_acc = s_acc + v
                            q_acc = q_acc + v * v
                            mn = jnp.minimum(mn, v)
                            mx = jnp.maximum(mx, v)
                        gmm_v[row, sl] = mn
                        gmm_v[row, C + cc * 16:C + (cc + 1) * 16] = mx
                        a = a_v[row // 2, pl.ds((row % 2) * C + cc * 16, 16)]
                        p_v[0, sl] = p_v[0, sl] + (K * a - s_acc)
                        p_v[1, sl] = (p_v[1, sl]
                                      + (K * (a * a) - 2.0 * (a * s_acc) + q_acc))
                    return _

                lax.fori_loop(0, rows_ck, row_body, 0)
            return _

        lax.fori_loop(0, chunks // nbuf, group_body, 0)

        pltpu.sync_copy(gmm_v, gmm_hbm.at[pl.ds(base, rows_w)])
        pltpu.sync_copy(p_v, part_hbm.at[wid])

    return run(edgeT2d, idx3d, aT2w)


def _final_body(cnt, aT_ref, gmm_ref, part_ref, gamma_ref, beta_ref, out_ref):
    p = jnp.sum(part_ref[...], axis=0)
    mean = p[0:1] / cnt
    var = p[1:2] / cnt - mean * mean
    inv = 1.0 / jnp.sqrt(var + EPS)
    s = gamma_ref[...] * inv
    bias = beta_ref[...] - mean * s

    C = aT_ref.shape[2]
    aT = aT_ref[0]
    gsel = jnp.where(s >= 0.0, gmm_ref[0, :, :C], gmm_ref[0, :, C:])
    y = (aT - gsel) * s + bias
    y = jnp.maximum(y, 0.0)
    out_ref[0] = jnp.transpose(y)


def _finalize(aT, gmm3d, part, gamma2d, beta2d):
    B, N, C = aT.shape
    nw = part.shape[0]
    nb = N // R
    cnt = float(B * N * K)
    return pl.pallas_call(
        functools.partial(_final_body, cnt),
        grid=(B, nb),
        in_specs=[
            pl.BlockSpec((1, R, C), lambda b, i: (b, i, 0)),
            pl.BlockSpec((1, R, 2 * C), lambda b, i: (b, i, 0)),
            pl.BlockSpec((nw, 2, C), lambda b, i: (0, 0, 0)),
            pl.BlockSpec((1, C), lambda b, i: (0, 0)),
            pl.BlockSpec((1, C), lambda b, i: (0, 0)),
        ],
        out_specs=pl.BlockSpec((1, C, R), lambda b, i: (b, 0, i)),
        out_shape=jax.ShapeDtypeStruct((B, C, N), F32),
        compiler_params=pltpu.CompilerParams(
            dimension_semantics=("parallel", "parallel")),
    )(aT, gmm3d, part, gamma2d, beta2d)


def kernel(features, W1, W2, gamma, beta):
    B, C, N = features.shape
    BN = B * N

    aT, edgeT, idx = _knn_topk(features, W1, W2)

    info = plsc.get_sparse_core_info()
    nw = info.num_cores * info.num_subcores
    rows_w = BN // nw
    rows_ck = 4
    chunks = rows_w // rows_ck
    idx3d = idx.reshape(nw, chunks, rows_ck * K)

    gmm, part = _sc_gather_stats(
        edgeT.reshape(BN, 128), idx3d, aT.reshape(BN // 2, 2 * C),
        nw, rows_w, chunks, rows_ck)

    return _finalize(aT, gmm.reshape(B, N, 2 * C), part,
                     gamma[None, :], beta[None, :])
